# baseline (speedup 1.0000x reference)
import functools

import jax
import jax.numpy as jnp
from jax import lax
from jax.experimental import pallas as pl
from jax.experimental.pallas import tpu as pltpu
from jax.experimental.pallas import tpu_sc as plsc

_NC = 2
_NS = 16
_NW = _NC * _NS

_F32 = jnp.float32
_CHUNK = 80
_DEGW = 16
_CH_ROWS = 400


def _row_chunks(n, si, fn):
    n_chunks = n // _CH_ROWS
    max_per_tile = (n_chunks + _NS - 1) // _NS
    for k in range(max_per_tile):
        cid = si + _NS * k
        if (k + 1) * _NS <= n_chunks:
            fn(cid * _CH_ROWS)
        else:
            @pl.when(cid < n_chunks)
            def _():
                fn(cid * _CH_ROWS)


def _sc_deg(n, iters, chunk):
    mesh = plsc.VectorSubcoreMesh(core_axis_name="c", subcore_axis_name="s")

    @functools.partial(
        pl.kernel,
        out_type=jax.ShapeDtypeStruct((_NC, n, _DEGW), _F32),
        mesh=mesh,
        compiler_params=pltpu.CompilerParams(use_tc_tiling_on_sc=False),
        scratch_types=[
            pltpu.VMEM((iters, chunk), jnp.int32),
            pltpu.VMEM((chunk, _DEGW), _F32),
            pltpu.VMEM_SHARED((n + _TRASH, _DEGW), _F32),
        ],
    )
    def deg(ones_hbm, zeros_hbm, dst_hbm, out_hbm, dst_v, ones_v, acc_sh):
        ci = lax.axis_index("c")
        si = lax.axis_index("s")
        wid = ci * _NS + si

        _row_chunks(n, si, lambda off: pltpu.sync_copy(
            zeros_hbm, acc_sh.at[pl.ds(off, _CH_ROWS)]))
        pltpu.sync_copy(ones_hbm, ones_v)
        pltpu.sync_copy(dst_hbm.at[wid], dst_v)
        plsc.subcore_barrier()

        def body(j, carry):
            pltpu.sync_copy(ones_v, acc_sh.at[dst_v.at[j]], add=True)
            return carry

        lax.fori_loop(0, iters, body, 0)
        plsc.subcore_barrier()

        def out_copy(off):
            pltpu.sync_copy(acc_sh.at[pl.ds(off, _CH_ROWS)],
                            out_hbm.at[ci, pl.ds(off, _CH_ROWS)])

        _row_chunks(n, si, out_copy)

    return deg


_NBUF = 5
_LAG = 1
_TRASH = 400


def _sc_agg(n, h, iters, chunk):
    mesh = plsc.VectorSubcoreMesh(core_axis_name="c", subcore_axis_name="s")

    @functools.partial(
        pl.kernel,
        out_type=jax.ShapeDtypeStruct((_NC, n, h), _F32),
        mesh=mesh,
        compiler_params=pltpu.CompilerParams(use_tc_tiling_on_sc=False),
        scratch_types=[
            pltpu.VMEM((iters, chunk), jnp.int32),
            pltpu.VMEM((iters, chunk), jnp.int32),
            pltpu.VMEM((_NBUF, chunk, h), _F32),
            pltpu.VMEM_SHARED((n + _TRASH, h), _F32),
        ] + [pltpu.SemaphoreType.DMA] * (2 * _NBUF),
    )
    def agg(g_hbm, src_hbm, dst_hbm, out_hbm,
            src_v, dst_v, rows_v, acc_sh, *sems):
        gsems = sems[:_NBUF]
        ssems = sems[_NBUF:]
        ci = lax.axis_index("c")
        si = lax.axis_index("s")
        wid = ci * _NS + si

        def init_copy(off):
            pltpu.sync_copy(g_hbm.at[pl.ds(off, _CH_ROWS)],
                            acc_sh.at[pl.ds(off, _CH_ROWS)])

        _row_chunks(n, si, init_copy)
        pltpu.sync_copy(src_hbm.at[wid], src_v)
        pltpu.sync_copy(dst_hbm.at[wid], dst_v)
        plsc.subcore_barrier()

        for b in range(_NBUF):
            pltpu.async_copy(g_hbm.at[src_v.at[b]], rows_v.at[b], gsems[b])

        def body(g, carry):
            for b in range(_NBUF):
                m = g * _NBUF + b
                pltpu.make_async_copy(
                    g_hbm.at[src_v.at[m]], rows_v.at[b], gsems[b]).wait()
                pltpu.async_copy(
                    rows_v.at[b], acc_sh.at[dst_v.at[m]], ssems[b], add=True)
                bp = (b - _LAG) % _NBUF
                mp = m - _LAG

                @pl.when(mp >= 0)
                def _():
                    pltpu.make_async_copy(
                        rows_v.at[bp], acc_sh.at[dst_v.at[mp]],
                        ssems[bp]).wait()

                    @pl.when(mp + _NBUF < iters)
                    def _():
                        pltpu.async_copy(
                            g_hbm.at[src_v.at[mp + _NBUF]], rows_v.at[bp],
                            gsems[bp])

            return carry

        lax.fori_loop(0, iters // _NBUF, body, 0)
        for k in range(_LAG):
            bl = (iters - _LAG + k) % _NBUF
            pltpu.make_async_copy(
                rows_v.at[bl], acc_sh.at[dst_v.at[iters - _LAG + k]],
                ssems[bl]).wait()
        plsc.subcore_barrier()

        def out_copy(off):
            pltpu.sync_copy(acc_sh.at[pl.ds(off, _CH_ROWS)],
                            out_hbm.at[ci, pl.ds(off, _CH_ROWS)])

        _row_chunks(n, si, out_copy)

    return agg


def _dot(a, b, precision=jax.lax.Precision.HIGHEST):
    return jax.lax.dot_general(a, b, (((1,), (0,)), ((), ())),
                               preferred_element_type=_F32,
                               precision=precision)


def _tc_first(deg_parts, x, w1, rb):
    n, d_in = x.shape
    hdim = w1.shape[1]

    def kfn(dp_ref, x_ref, w_ref, dinv_ref, g_ref):
        dp = dp_ref[0] + dp_ref[1]
        deg = jnp.sum(dp, axis=1, keepdims=True) + 1.0
        dinv = lax.rsqrt(deg)
        dinv_ref[...] = dinv
        g_ref[...] = dinv * _dot(x_ref[...], w_ref[...])

    return pl.pallas_call(
        kfn,
        grid=(n // rb,),
        in_specs=[
            pl.BlockSpec((2, rb, _DEGW), lambda i: (0, i, 0)),
            pl.BlockSpec((rb, d_in), lambda i: (i, 0)),
            pl.BlockSpec((d_in, hdim), lambda i: (0, 0)),
        ],
        out_specs=[
            pl.BlockSpec((rb, 1), lambda i: (i, 0)),
            pl.BlockSpec((rb, hdim), lambda i: (i, 0)),
        ],
        out_shape=[
            jax.ShapeDtypeStruct((n, 1), _F32),
            jax.ShapeDtypeStruct((n, hdim), _F32),
        ],
    )(deg_parts, x, w1)


def _tc_mid(s_parts, g_prev, dinv, w_next, b, rb):
    n, hdim = g_prev.shape
    hout = w_next.shape[1]

    def kfn(sp_ref, gp_ref, dinv_ref, w_ref, b_ref, out_ref):
        s = sp_ref[0] + sp_ref[1] - gp_ref[...]
        dv = dinv_ref[...]
        hact = jnp.maximum(dv * s + b_ref[...], 0.0)
        out_ref[...] = dv * _dot(hact, w_ref[...])

    return pl.pallas_call(
        kfn,
        grid=(n // rb,),
        in_specs=[
            pl.BlockSpec((2, rb, hdim), lambda i: (0, i, 0)),
            pl.BlockSpec((rb, hdim), lambda i: (i, 0)),
            pl.BlockSpec((rb, 1), lambda i: (i, 0)),
            pl.BlockSpec((hdim, hout), lambda i: (0, 0)),
            pl.BlockSpec((1, hdim), lambda i: (0, 0)),
        ],
        out_specs=pl.BlockSpec((rb, hout), lambda i: (i, 0)),
        out_shape=jax.ShapeDtypeStruct((n, hout), _F32),
    )(s_parts, g_prev, dinv, w_next, b)


def _tc_last_g(s_parts, g_prev, dinv, b, rb):
    n, hdim = g_prev.shape

    def kfn(sp_ref, gp_ref, dinv_ref, b_ref, out_ref):
        s = sp_ref[0] + sp_ref[1] - gp_ref[...]
        dv = dinv_ref[...]
        out_ref[...] = dv * jnp.maximum(dv * s + b_ref[...], 0.0)

    return pl.pallas_call(
        kfn,
        grid=(n // rb,),
        in_specs=[
            pl.BlockSpec((2, rb, hdim), lambda i: (0, i, 0)),
            pl.BlockSpec((rb, hdim), lambda i: (i, 0)),
            pl.BlockSpec((rb, 1), lambda i: (i, 0)),
            pl.BlockSpec((1, hdim), lambda i: (0, 0)),
        ],
        out_specs=pl.BlockSpec((rb, hdim), lambda i: (i, 0)),
        out_shape=jax.ShapeDtypeStruct((n, hdim), _F32),
    )(s_parts, g_prev, dinv, b)


def _tc_readout(s_parts, g_prev, dinv, w4, b4, batch3, nseg, rb):
    n, hdim = g_prev.shape
    dout = w4.shape[1]
    nblocks = n // rb

    def kfn(sp_ref, gp_ref, dinv_ref, w_ref, b_ref, batch_ref, out_ref,
            sums_sc, cnt_sc):
        i = pl.program_id(0)

        @pl.when(i == 0)
        def _():
            sums_sc[...] = jnp.zeros_like(sums_sc)
            cnt_sc[...] = jnp.zeros_like(cnt_sc)

        u = dinv_ref[...] * (sp_ref[0] + sp_ref[1] - gp_ref[...])
        h4 = _dot(u, w_ref[...]) + b_ref[...]
        bvec = batch_ref[0]
        onehot_t = jnp.where(
            lax.broadcasted_iota(jnp.int32, (nseg, rb), 0) == bvec, 1.0, 0.0)
        sums_sc[...] += _dot(onehot_t, h4)
        cnt_sc[...] += jnp.sum(onehot_t, axis=1, keepdims=True)

        @pl.when(i == nblocks - 1)
        def _():
            mean = sums_sc[...] / jnp.maximum(cnt_sc[...], 1.0)
            m = jnp.max(mean, axis=1, keepdims=True)
            lse = jnp.log(jnp.sum(jnp.exp(mean - m), axis=1, keepdims=True)) + m
            out_ref[...] = mean - lse

    return pl.pallas_call(
        kfn,
        grid=(nblocks,),
        in_specs=[
            pl.BlockSpec((2, rb, hdim), lambda i: (0, i, 0)),
            pl.BlockSpec((rb, hdim), lambda i: (i, 0)),
            pl.BlockSpec((rb, 1), lambda i: (i, 0)),
            pl.BlockSpec((hdim, dout), lambda i: (0, 0)),
            pl.BlockSpec((1, dout), lambda i: (0, 0)),
            pl.BlockSpec((1, 1, rb), lambda i: (i, 0, 0)),
        ],
        out_specs=pl.BlockSpec((nseg, dout), lambda i: (0, 0)),
        out_shape=jax.ShapeDtypeStruct((nseg, dout), _F32),
        scratch_shapes=[
            pltpu.VMEM((nseg, dout), _F32),
            pltpu.VMEM((nseg, 1), _F32),
        ],
    )(s_parts, g_prev, dinv, w4, b4, batch3)


def kernel(x, edge_index, batch, W1, b1, W2, b2, W3, b3, W4, b4):
    n, d_in = x.shape
    e = edge_index.shape[1]
    hdim = W1.shape[1]
    nseg = 64
    rb = 5000

    per_w = e // _NW
    iters = -(-per_w // (_CHUNK * _NBUF)) * _NBUF
    e_pad = _NW * iters * _CHUNK - e
    if e_pad:
        src_p = jnp.concatenate([edge_index[0],
                                 jnp.zeros((e_pad,), jnp.int32)])
        dst_p = jnp.concatenate([edge_index[1],
                                 n + (jnp.arange(e_pad, dtype=jnp.int32)
                                      % _TRASH)])
    else:
        src_p, dst_p = edge_index[0], edge_index[1]
    src3 = src_p.reshape(_NW, iters, _CHUNK)
    dst3 = dst_p.reshape(_NW, iters, _CHUNK)
    ones_h = jnp.ones((_CHUNK, _DEGW), _F32)
    zeros_h = jnp.zeros((_CH_ROWS, _DEGW), _F32)
    batch3 = batch.reshape(n // rb, 1, rb)

    deg_parts = _sc_deg(n, iters, _CHUNK)(ones_h, zeros_h, dst3)
    dinv, g1 = _tc_first(deg_parts, x, W1, rb)

    agg = _sc_agg(n, hdim, iters, _CHUNK)
    s1 = agg(g1, src3, dst3)
    g2 = _tc_mid(s1, g1, dinv, W2, b1.reshape(1, -1), rb)
    s2 = agg(g2, src3, dst3)
    g3 = _tc_mid(s2, g2, dinv, W3, b2.reshape(1, -1), rb)
    s3 = agg(g3, src3, dst3)
    g4 = _tc_last_g(s3, g3, dinv, b3.reshape(1, -1), rb)
    s4 = agg(g4, src3, dst3)
    return _tc_readout(s4, g4, dinv, W4, b4.reshape(1, -1), batch3, nseg, rb)


# speedup vs baseline: 1.0293x; 1.0293x over previous
import functools

import jax
import jax.numpy as jnp
from jax import lax
from jax.experimental import pallas as pl
from jax.experimental.pallas import tpu as pltpu
from jax.experimental.pallas import tpu_sc as plsc

_NC = 2
_NS = 16
_NW = _NC * _NS

_F32 = jnp.float32
_CHUNK = 80
_DEGW = 8
_CH_ROWS = 400


def _row_chunks(n, si, fn):
    n_chunks = n // _CH_ROWS
    max_per_tile = (n_chunks + _NS - 1) // _NS
    for k in range(max_per_tile):
        cid = si + _NS * k
        if (k + 1) * _NS <= n_chunks:
            fn(cid * _CH_ROWS)
        else:
            @pl.when(cid < n_chunks)
            def _():
                fn(cid * _CH_ROWS)


def _sc_deg(n, iters, chunk):
    mesh = plsc.VectorSubcoreMesh(core_axis_name="c", subcore_axis_name="s")

    @functools.partial(
        pl.kernel,
        out_type=jax.ShapeDtypeStruct((_NC, n, _DEGW), _F32),
        mesh=mesh,
        compiler_params=pltpu.CompilerParams(use_tc_tiling_on_sc=False),
        scratch_types=[
            pltpu.VMEM((iters, chunk), jnp.int32),
            pltpu.VMEM((chunk, _DEGW), _F32),
            pltpu.VMEM_SHARED((n + _TRASH, _DEGW), _F32),
        ],
    )
    def deg(ones_hbm, zeros_hbm, dst_hbm, out_hbm, dst_v, ones_v, acc_sh):
        ci = lax.axis_index("c")
        si = lax.axis_index("s")
        wid = ci * _NS + si

        _row_chunks(n, si, lambda off: pltpu.sync_copy(
            zeros_hbm, acc_sh.at[pl.ds(off, _CH_ROWS)]))
        pltpu.sync_copy(ones_hbm, ones_v)
        pltpu.sync_copy(dst_hbm.at[wid], dst_v)
        plsc.subcore_barrier()

        def body(j, carry):
            pltpu.sync_copy(ones_v, acc_sh.at[dst_v.at[j]], add=True)
            return carry

        lax.fori_loop(0, iters, body, 0)
        plsc.subcore_barrier()

        def out_copy(off):
            pltpu.sync_copy(acc_sh.at[pl.ds(off, _CH_ROWS)],
                            out_hbm.at[ci, pl.ds(off, _CH_ROWS)])

        _row_chunks(n, si, out_copy)

    return deg


_NBUF = 5
_LAG = 1
_TRASH = 400


def _sc_agg(n, h, iters, chunk):
    mesh = plsc.VectorSubcoreMesh(core_axis_name="c", subcore_axis_name="s")

    @functools.partial(
        pl.kernel,
        out_type=jax.ShapeDtypeStruct((_NC, n, h), _F32),
        mesh=mesh,
        compiler_params=pltpu.CompilerParams(use_tc_tiling_on_sc=False),
        scratch_types=[
            pltpu.VMEM((iters, chunk), jnp.int32),
            pltpu.VMEM((iters, chunk), jnp.int32),
            pltpu.VMEM((_NBUF, chunk, h), _F32),
            pltpu.VMEM_SHARED((n + _TRASH, h), _F32),
        ] + [pltpu.SemaphoreType.DMA] * (2 * _NBUF),
    )
    def agg(g_hbm, src_hbm, dst_hbm, out_hbm,
            src_v, dst_v, rows_v, acc_sh, *sems):
        gsems = sems[:_NBUF]
        ssems = sems[_NBUF:]
        ci = lax.axis_index("c")
        si = lax.axis_index("s")
        wid = ci * _NS + si

        def init_copy(off):
            pltpu.sync_copy(g_hbm.at[pl.ds(off, _CH_ROWS)],
                            acc_sh.at[pl.ds(off, _CH_ROWS)])

        _row_chunks(n, si, init_copy)
        pltpu.sync_copy(src_hbm.at[wid], src_v)
        pltpu.sync_copy(dst_hbm.at[wid], dst_v)
        plsc.subcore_barrier()

        for b in range(_NBUF):
            pltpu.async_copy(g_hbm.at[src_v.at[b]], rows_v.at[b], gsems[b])

        def body(g, carry):
            for b in range(_NBUF):
                m = g * _NBUF + b
                pltpu.make_async_copy(
                    g_hbm.at[src_v.at[m]], rows_v.at[b], gsems[b]).wait()
                pltpu.async_copy(
                    rows_v.at[b], acc_sh.at[dst_v.at[m]], ssems[b], add=True)
                bp = (b - _LAG) % _NBUF
                mp = m - _LAG

                @pl.when(mp >= 0)
                def _():
                    pltpu.make_async_copy(
                        rows_v.at[bp], acc_sh.at[dst_v.at[mp]],
                        ssems[bp]).wait()

                    @pl.when(mp + _NBUF < iters)
                    def _():
                        pltpu.async_copy(
                            g_hbm.at[src_v.at[mp + _NBUF]], rows_v.at[bp],
                            gsems[bp])

            return carry

        lax.fori_loop(0, iters // _NBUF, body, 0)
        for k in range(_LAG):
            bl = (iters - _LAG + k) % _NBUF
            pltpu.make_async_copy(
                rows_v.at[bl], acc_sh.at[dst_v.at[iters - _LAG + k]],
                ssems[bl]).wait()
        plsc.subcore_barrier()

        def out_copy(off):
            pltpu.sync_copy(acc_sh.at[pl.ds(off, _CH_ROWS)],
                            out_hbm.at[ci, pl.ds(off, _CH_ROWS)])

        _row_chunks(n, si, out_copy)

    return agg


def _dot(a, b, precision=jax.lax.Precision.HIGHEST):
    return jax.lax.dot_general(a, b, (((1,), (0,)), ((), ())),
                               preferred_element_type=_F32,
                               precision=precision)


def _tc_first(deg_parts, x, w1, rb):
    n, d_in = x.shape
    hdim = w1.shape[1]

    def kfn(dp_ref, x_ref, w_ref, dinv_ref, g_ref):
        dp = dp_ref[0] + dp_ref[1]
        deg = jnp.sum(dp, axis=1, keepdims=True) + 1.0
        dinv = lax.rsqrt(deg)
        dinv_ref[...] = dinv
        g_ref[...] = dinv * _dot(x_ref[...], w_ref[...])

    return pl.pallas_call(
        kfn,
        grid=(n // rb,),
        in_specs=[
            pl.BlockSpec((2, rb, _DEGW), lambda i: (0, i, 0)),
            pl.BlockSpec((rb, d_in), lambda i: (i, 0)),
            pl.BlockSpec((d_in, hdim), lambda i: (0, 0)),
        ],
        out_specs=[
            pl.BlockSpec((rb, 1), lambda i: (i, 0)),
            pl.BlockSpec((rb, hdim), lambda i: (i, 0)),
        ],
        out_shape=[
            jax.ShapeDtypeStruct((n, 1), _F32),
            jax.ShapeDtypeStruct((n, hdim), _F32),
        ],
    )(deg_parts, x, w1)


def _tc_mid(s_parts, g_prev, dinv, w_next, b, rb):
    n, hdim = g_prev.shape
    hout = w_next.shape[1]

    def kfn(sp_ref, gp_ref, dinv_ref, w_ref, b_ref, out_ref):
        s = sp_ref[0] + sp_ref[1] - gp_ref[...]
        dv = dinv_ref[...]
        hact = jnp.maximum(dv * s + b_ref[...], 0.0)
        out_ref[...] = dv * _dot(hact, w_ref[...])

    return pl.pallas_call(
        kfn,
        grid=(n // rb,),
        in_specs=[
            pl.BlockSpec((2, rb, hdim), lambda i: (0, i, 0)),
            pl.BlockSpec((rb, hdim), lambda i: (i, 0)),
            pl.BlockSpec((rb, 1), lambda i: (i, 0)),
            pl.BlockSpec((hdim, hout), lambda i: (0, 0)),
            pl.BlockSpec((1, hdim), lambda i: (0, 0)),
        ],
        out_specs=pl.BlockSpec((rb, hout), lambda i: (i, 0)),
        out_shape=jax.ShapeDtypeStruct((n, hout), _F32),
    )(s_parts, g_prev, dinv, w_next, b)


def _tc_last_g(s_parts, g_prev, dinv, b, rb):
    n, hdim = g_prev.shape

    def kfn(sp_ref, gp_ref, dinv_ref, b_ref, out_ref):
        s = sp_ref[0] + sp_ref[1] - gp_ref[...]
        dv = dinv_ref[...]
        out_ref[...] = dv * jnp.maximum(dv * s + b_ref[...], 0.0)

    return pl.pallas_call(
        kfn,
        grid=(n // rb,),
        in_specs=[
            pl.BlockSpec((2, rb, hdim), lambda i: (0, i, 0)),
            pl.BlockSpec((rb, hdim), lambda i: (i, 0)),
            pl.BlockSpec((rb, 1), lambda i: (i, 0)),
            pl.BlockSpec((1, hdim), lambda i: (0, 0)),
        ],
        out_specs=pl.BlockSpec((rb, hdim), lambda i: (i, 0)),
        out_shape=jax.ShapeDtypeStruct((n, hdim), _F32),
    )(s_parts, g_prev, dinv, b)


def _tc_readout(s_parts, g_prev, dinv, w4, b4, batch3, nseg, rb):
    n, hdim = g_prev.shape
    dout = w4.shape[1]
    nblocks = n // rb

    def kfn(sp_ref, gp_ref, dinv_ref, w_ref, b_ref, batch_ref, out_ref,
            sums_sc, cnt_sc):
        i = pl.program_id(0)

        @pl.when(i == 0)
        def _():
            sums_sc[...] = jnp.zeros_like(sums_sc)
            cnt_sc[...] = jnp.zeros_like(cnt_sc)

        u = dinv_ref[...] * (sp_ref[0] + sp_ref[1] - gp_ref[...])
        h4 = _dot(u, w_ref[...]) + b_ref[...]
        bvec = batch_ref[0]
        onehot_t = jnp.where(
            lax.broadcasted_iota(jnp.int32, (nseg, rb), 0) == bvec, 1.0, 0.0)
        sums_sc[...] += _dot(onehot_t, h4)
        cnt_sc[...] += jnp.sum(onehot_t, axis=1, keepdims=True)

        @pl.when(i == nblocks - 1)
        def _():
            mean = sums_sc[...] / jnp.maximum(cnt_sc[...], 1.0)
            m = jnp.max(mean, axis=1, keepdims=True)
            lse = jnp.log(jnp.sum(jnp.exp(mean - m), axis=1, keepdims=True)) + m
            out_ref[...] = mean - lse

    return pl.pallas_call(
        kfn,
        grid=(nblocks,),
        in_specs=[
            pl.BlockSpec((2, rb, hdim), lambda i: (0, i, 0)),
            pl.BlockSpec((rb, hdim), lambda i: (i, 0)),
            pl.BlockSpec((rb, 1), lambda i: (i, 0)),
            pl.BlockSpec((hdim, dout), lambda i: (0, 0)),
            pl.BlockSpec((1, dout), lambda i: (0, 0)),
            pl.BlockSpec((1, 1, rb), lambda i: (i, 0, 0)),
        ],
        out_specs=pl.BlockSpec((nseg, dout), lambda i: (0, 0)),
        out_shape=jax.ShapeDtypeStruct((nseg, dout), _F32),
        scratch_shapes=[
            pltpu.VMEM((nseg, dout), _F32),
            pltpu.VMEM((nseg, 1), _F32),
        ],
    )(s_parts, g_prev, dinv, w4, b4, batch3)


def kernel(x, edge_index, batch, W1, b1, W2, b2, W3, b3, W4, b4):
    n, d_in = x.shape
    e = edge_index.shape[1]
    hdim = W1.shape[1]
    nseg = 64
    rb = 2000

    per_w = e // _NW
    iters = -(-per_w // (_CHUNK * _NBUF)) * _NBUF
    e_pad = _NW * iters * _CHUNK - e
    if e_pad:
        src_p = jnp.concatenate([edge_index[0],
                                 jnp.zeros((e_pad,), jnp.int32)])
        dst_p = jnp.concatenate([edge_index[1],
                                 n + (jnp.arange(e_pad, dtype=jnp.int32)
                                      % _TRASH)])
    else:
        src_p, dst_p = edge_index[0], edge_index[1]
    src3 = src_p.reshape(_NW, iters, _CHUNK)
    dst3 = dst_p.reshape(_NW, iters, _CHUNK)
    ones_h = jnp.ones((_CHUNK, _DEGW), _F32)
    zeros_h = jnp.zeros((_CH_ROWS, _DEGW), _F32)
    batch3 = batch.reshape(n // rb, 1, rb)

    deg_parts = _sc_deg(n, iters, _CHUNK)(ones_h, zeros_h, dst3)
    dinv, g1 = _tc_first(deg_parts, x, W1, rb)

    agg = _sc_agg(n, hdim, iters, _CHUNK)
    s1 = agg(g1, src3, dst3)
    g2 = _tc_mid(s1, g1, dinv, W2, b1.reshape(1, -1), rb)
    s2 = agg(g2, src3, dst3)
    g3 = _tc_mid(s2, g2, dinv, W3, b2.reshape(1, -1), rb)
    s3 = agg(g3, src3, dst3)
    g4 = _tc_last_g(s3, g3, dinv, b3.reshape(1, -1), rb)
    s4 = agg(g4, src3, dst3)
    return _tc_readout(s4, g4, dinv, W4, b4.reshape(1, -1), batch3, nseg, rb)


# speedup vs baseline: 1.0470x; 1.0171x over previous
import functools

import jax
import jax.numpy as jnp
from jax import lax
from jax.experimental import pallas as pl
from jax.experimental.pallas import tpu as pltpu
from jax.experimental.pallas import tpu_sc as plsc

_NC = 2
_NS = 16
_NW = _NC * _NS

_F32 = jnp.float32
_CHUNK = 80
_DEGW = 8
_CH_ROWS = 400


def _row_chunks(n, si, fn):
    n_chunks = n // _CH_ROWS
    max_per_tile = (n_chunks + _NS - 1) // _NS
    for k in range(max_per_tile):
        cid = si + _NS * k
        if (k + 1) * _NS <= n_chunks:
            fn(cid * _CH_ROWS)
        else:
            @pl.when(cid < n_chunks)
            def _():
                fn(cid * _CH_ROWS)


def _sc_deg(n, iters, chunk):
    mesh = plsc.VectorSubcoreMesh(core_axis_name="c", subcore_axis_name="s")

    @functools.partial(
        pl.kernel,
        out_type=jax.ShapeDtypeStruct((_NC, n, _DEGW), _F32),
        mesh=mesh,
        compiler_params=pltpu.CompilerParams(use_tc_tiling_on_sc=False),
        scratch_types=[
            pltpu.VMEM((iters, chunk), jnp.int32),
            pltpu.VMEM((chunk, _DEGW), _F32),
            pltpu.VMEM_SHARED((n + _TRASH, _DEGW), _F32),
        ] + [pltpu.SemaphoreType.DMA] * _NBUF,
    )
    def deg(ones_hbm, zeros_hbm, dst_hbm, out_hbm, dst_v, ones_v, acc_sh,
            *ssems):
        ci = lax.axis_index("c")
        si = lax.axis_index("s")
        wid = ci * _NS + si

        _row_chunks(n, si, lambda off: pltpu.sync_copy(
            zeros_hbm, acc_sh.at[pl.ds(off, _CH_ROWS)]))
        pltpu.sync_copy(ones_hbm, ones_v)
        pltpu.sync_copy(dst_hbm.at[wid], dst_v)
        plsc.subcore_barrier()

        for b in range(_NBUF):
            pltpu.async_copy(ones_v, acc_sh.at[dst_v.at[b]], ssems[b],
                             add=True)

        def body(g, carry):
            for b in range(_NBUF):
                m = (g + 1) * _NBUF + b
                pltpu.make_async_copy(
                    ones_v, acc_sh.at[dst_v.at[m - _NBUF]], ssems[b]).wait()
                pltpu.async_copy(ones_v, acc_sh.at[dst_v.at[m]], ssems[b],
                                 add=True)
            return carry

        lax.fori_loop(0, iters // _NBUF - 1, body, 0)
        for b in range(_NBUF):
            pltpu.make_async_copy(
                ones_v, acc_sh.at[dst_v.at[iters - _NBUF + b]],
                ssems[b]).wait()
        plsc.subcore_barrier()

        def out_copy(off):
            pltpu.sync_copy(acc_sh.at[pl.ds(off, _CH_ROWS)],
                            out_hbm.at[ci, pl.ds(off, _CH_ROWS)])

        _row_chunks(n, si, out_copy)

    return deg


_NBUF = 5
_LAG = 1
_TRASH = 400


def _sc_agg(n, h, iters, chunk):
    mesh = plsc.VectorSubcoreMesh(core_axis_name="c", subcore_axis_name="s")

    @functools.partial(
        pl.kernel,
        out_type=jax.ShapeDtypeStruct((_NC, n, h), _F32),
        mesh=mesh,
        compiler_params=pltpu.CompilerParams(use_tc_tiling_on_sc=False),
        scratch_types=[
            pltpu.VMEM((iters, chunk), jnp.int32),
            pltpu.VMEM((iters, chunk), jnp.int32),
            pltpu.VMEM((_NBUF, chunk, h), _F32),
            pltpu.VMEM_SHARED((n + _TRASH, h), _F32),
        ] + [pltpu.SemaphoreType.DMA] * (2 * _NBUF),
    )
    def agg(g_hbm, src_hbm, dst_hbm, out_hbm,
            src_v, dst_v, rows_v, acc_sh, *sems):
        gsems = sems[:_NBUF]
        ssems = sems[_NBUF:]
        ci = lax.axis_index("c")
        si = lax.axis_index("s")
        wid = ci * _NS + si

        def init_copy(off):
            pltpu.sync_copy(g_hbm.at[pl.ds(off, _CH_ROWS)],
                            acc_sh.at[pl.ds(off, _CH_ROWS)])

        _row_chunks(n, si, init_copy)
        pltpu.sync_copy(src_hbm.at[wid], src_v)
        pltpu.sync_copy(dst_hbm.at[wid], dst_v)
        plsc.subcore_barrier()

        for b in range(_NBUF):
            pltpu.async_copy(g_hbm.at[src_v.at[b]], rows_v.at[b], gsems[b])

        def body(g, carry):
            for b in range(_NBUF):
                m = g * _NBUF + b
                pltpu.make_async_copy(
                    g_hbm.at[src_v.at[m]], rows_v.at[b], gsems[b]).wait()
                pltpu.async_copy(
                    rows_v.at[b], acc_sh.at[dst_v.at[m]], ssems[b], add=True)
                bp = (b - _LAG) % _NBUF
                mp = m - _LAG

                @pl.when(mp >= 0)
                def _():
                    pltpu.make_async_copy(
                        rows_v.at[bp], acc_sh.at[dst_v.at[mp]],
                        ssems[bp]).wait()

                    @pl.when(mp + _NBUF < iters)
                    def _():
                        pltpu.async_copy(
                            g_hbm.at[src_v.at[mp + _NBUF]], rows_v.at[bp],
                            gsems[bp])

            return carry

        lax.fori_loop(0, iters // _NBUF, body, 0)
        for k in range(_LAG):
            bl = (iters - _LAG + k) % _NBUF
            pltpu.make_async_copy(
                rows_v.at[bl], acc_sh.at[dst_v.at[iters - _LAG + k]],
                ssems[bl]).wait()
        plsc.subcore_barrier()

        def out_copy(off):
            pltpu.sync_copy(acc_sh.at[pl.ds(off, _CH_ROWS)],
                            out_hbm.at[ci, pl.ds(off, _CH_ROWS)])

        _row_chunks(n, si, out_copy)

    return agg


def _dot(a, b, precision=jax.lax.Precision.HIGHEST):
    return jax.lax.dot_general(a, b, (((1,), (0,)), ((), ())),
                               preferred_element_type=_F32,
                               precision=precision)


def _tc_first(deg_parts, x, w1, rb):
    n, d_in = x.shape
    hdim = w1.shape[1]

    def kfn(dp_ref, x_ref, w_ref, dinv_ref, g_ref):
        dp = dp_ref[0] + dp_ref[1]
        deg = jnp.sum(dp, axis=1, keepdims=True) + 1.0
        dinv = lax.rsqrt(deg)
        dinv_ref[...] = dinv
        g_ref[...] = dinv * _dot(x_ref[...], w_ref[...])

    return pl.pallas_call(
        kfn,
        grid=(n // rb,),
        in_specs=[
            pl.BlockSpec((2, rb, _DEGW), lambda i: (0, i, 0)),
            pl.BlockSpec((rb, d_in), lambda i: (i, 0)),
            pl.BlockSpec((d_in, hdim), lambda i: (0, 0)),
        ],
        out_specs=[
            pl.BlockSpec((rb, 1), lambda i: (i, 0)),
            pl.BlockSpec((rb, hdim), lambda i: (i, 0)),
        ],
        out_shape=[
            jax.ShapeDtypeStruct((n, 1), _F32),
            jax.ShapeDtypeStruct((n, hdim), _F32),
        ],
    )(deg_parts, x, w1)


def _tc_mid(s_parts, g_prev, dinv, w_next, b, rb):
    n, hdim = g_prev.shape
    hout = w_next.shape[1]

    def kfn(sp_ref, gp_ref, dinv_ref, w_ref, b_ref, out_ref):
        s = sp_ref[0] + sp_ref[1] - gp_ref[...]
        dv = dinv_ref[...]
        hact = jnp.maximum(dv * s + b_ref[...], 0.0)
        out_ref[...] = dv * _dot(hact, w_ref[...])

    return pl.pallas_call(
        kfn,
        grid=(n // rb,),
        in_specs=[
            pl.BlockSpec((2, rb, hdim), lambda i: (0, i, 0)),
            pl.BlockSpec((rb, hdim), lambda i: (i, 0)),
            pl.BlockSpec((rb, 1), lambda i: (i, 0)),
            pl.BlockSpec((hdim, hout), lambda i: (0, 0)),
            pl.BlockSpec((1, hdim), lambda i: (0, 0)),
        ],
        out_specs=pl.BlockSpec((rb, hout), lambda i: (i, 0)),
        out_shape=jax.ShapeDtypeStruct((n, hout), _F32),
    )(s_parts, g_prev, dinv, w_next, b)


def _tc_last_g(s_parts, g_prev, dinv, b, rb):
    n, hdim = g_prev.shape

    def kfn(sp_ref, gp_ref, dinv_ref, b_ref, out_ref):
        s = sp_ref[0] + sp_ref[1] - gp_ref[...]
        dv = dinv_ref[...]
        out_ref[...] = dv * jnp.maximum(dv * s + b_ref[...], 0.0)

    return pl.pallas_call(
        kfn,
        grid=(n // rb,),
        in_specs=[
            pl.BlockSpec((2, rb, hdim), lambda i: (0, i, 0)),
            pl.BlockSpec((rb, hdim), lambda i: (i, 0)),
            pl.BlockSpec((rb, 1), lambda i: (i, 0)),
            pl.BlockSpec((1, hdim), lambda i: (0, 0)),
        ],
        out_specs=pl.BlockSpec((rb, hdim), lambda i: (i, 0)),
        out_shape=jax.ShapeDtypeStruct((n, hdim), _F32),
    )(s_parts, g_prev, dinv, b)


def _tc_readout(s_parts, g_prev, dinv, w4, b4, batch3, nseg, rb):
    n, hdim = g_prev.shape
    dout = w4.shape[1]
    nblocks = n // rb

    def kfn(sp_ref, gp_ref, dinv_ref, w_ref, b_ref, batch_ref, out_ref,
            sums_sc, cnt_sc):
        i = pl.program_id(0)

        @pl.when(i == 0)
        def _():
            sums_sc[...] = jnp.zeros_like(sums_sc)
            cnt_sc[...] = jnp.zeros_like(cnt_sc)

        u = dinv_ref[...] * (sp_ref[0] + sp_ref[1] - gp_ref[...])
        h4 = _dot(u, w_ref[...]) + b_ref[...]
        bvec = batch_ref[0]
        onehot_t = jnp.where(
            lax.broadcasted_iota(jnp.int32, (nseg, rb), 0) == bvec, 1.0, 0.0)
        sums_sc[...] += _dot(onehot_t, h4)
        cnt_sc[...] += jnp.sum(onehot_t, axis=1, keepdims=True)

        @pl.when(i == nblocks - 1)
        def _():
            mean = sums_sc[...] / jnp.maximum(cnt_sc[...], 1.0)
            m = jnp.max(mean, axis=1, keepdims=True)
            lse = jnp.log(jnp.sum(jnp.exp(mean - m), axis=1, keepdims=True)) + m
            out_ref[...] = mean - lse

    return pl.pallas_call(
        kfn,
        grid=(nblocks,),
        in_specs=[
            pl.BlockSpec((2, rb, hdim), lambda i: (0, i, 0)),
            pl.BlockSpec((rb, hdim), lambda i: (i, 0)),
            pl.BlockSpec((rb, 1), lambda i: (i, 0)),
            pl.BlockSpec((hdim, dout), lambda i: (0, 0)),
            pl.BlockSpec((1, dout), lambda i: (0, 0)),
            pl.BlockSpec((1, 1, rb), lambda i: (i, 0, 0)),
        ],
        out_specs=pl.BlockSpec((nseg, dout), lambda i: (0, 0)),
        out_shape=jax.ShapeDtypeStruct((nseg, dout), _F32),
        scratch_shapes=[
            pltpu.VMEM((nseg, dout), _F32),
            pltpu.VMEM((nseg, 1), _F32),
        ],
    )(s_parts, g_prev, dinv, w4, b4, batch3)


def kernel(x, edge_index, batch, W1, b1, W2, b2, W3, b3, W4, b4):
    n, d_in = x.shape
    e = edge_index.shape[1]
    hdim = W1.shape[1]
    nseg = 64
    rb = 2000

    per_w = e // _NW
    iters = -(-per_w // (_CHUNK * _NBUF)) * _NBUF
    e_pad = _NW * iters * _CHUNK - e
    if e_pad:
        src_p = jnp.concatenate([edge_index[0],
                                 jnp.zeros((e_pad,), jnp.int32)])
        dst_p = jnp.concatenate([edge_index[1],
                                 n + (jnp.arange(e_pad, dtype=jnp.int32)
                                      % _TRASH)])
    else:
        src_p, dst_p = edge_index[0], edge_index[1]
    src3 = src_p.reshape(_NW, iters, _CHUNK)
    dst3 = dst_p.reshape(_NW, iters, _CHUNK)
    ones_h = jnp.ones((_CHUNK, _DEGW), _F32)
    zeros_h = jnp.zeros((_CH_ROWS, _DEGW), _F32)
    batch3 = batch.reshape(n // rb, 1, rb)

    deg_parts = _sc_deg(n, iters, _CHUNK)(ones_h, zeros_h, dst3)
    dinv, g1 = _tc_first(deg_parts, x, W1, rb)

    agg = _sc_agg(n, hdim, iters, _CHUNK)
    s1 = agg(g1, src3, dst3)
    g2 = _tc_mid(s1, g1, dinv, W2, b1.reshape(1, -1), rb)
    s2 = agg(g2, src3, dst3)
    g3 = _tc_mid(s2, g2, dinv, W3, b2.reshape(1, -1), rb)
    s3 = agg(g3, src3, dst3)
    g4 = _tc_last_g(s3, g3, dinv, b3.reshape(1, -1), rb)
    s4 = agg(g4, src3, dst3)
    return _tc_readout(s4, g4, dinv, W4, b4.reshape(1, -1), batch3, nseg, rb)


# speedup vs baseline: 1.0825x; 1.0339x over previous
import functools

import jax
import jax.numpy as jnp
from jax import lax
from jax.experimental import pallas as pl
from jax.experimental.pallas import tpu as pltpu
from jax.experimental.pallas import tpu_sc as plsc

_NC = 2
_NS = 16
_NW = _NC * _NS

_F32 = jnp.float32
_CHUNK = 80
_DEGW = 8
_CH_ROWS = 400


def _row_chunks(n, si, fn):
    n_chunks = n // _CH_ROWS
    max_per_tile = (n_chunks + _NS - 1) // _NS
    for k in range(max_per_tile):
        cid = si + _NS * k
        if (k + 1) * _NS <= n_chunks:
            fn(cid * _CH_ROWS)
        else:
            @pl.when(cid < n_chunks)
            def _():
                fn(cid * _CH_ROWS)


def _sc_deg(n, iters, chunk):
    mesh = plsc.VectorSubcoreMesh(core_axis_name="c", subcore_axis_name="s")

    @functools.partial(
        pl.kernel,
        out_type=jax.ShapeDtypeStruct((_NC, n, _DEGW), _F32),
        mesh=mesh,
        compiler_params=pltpu.CompilerParams(use_tc_tiling_on_sc=False),
        scratch_types=[
            pltpu.VMEM((iters, chunk), jnp.int32),
            pltpu.VMEM((chunk, _DEGW), _F32),
            pltpu.VMEM_SHARED((n + _TRASH, _DEGW), _F32),
        ] + [pltpu.SemaphoreType.DMA] * _NBUF,
    )
    def deg(ones_hbm, zeros_hbm, dst_hbm, out_hbm, dst_v, ones_v, acc_sh,
            *ssems):
        ci = lax.axis_index("c")
        si = lax.axis_index("s")
        wid = ci * _NS + si

        _row_chunks(n, si, lambda off: pltpu.sync_copy(
            zeros_hbm, acc_sh.at[pl.ds(off, _CH_ROWS)]))
        pltpu.sync_copy(ones_hbm, ones_v)
        pltpu.sync_copy(dst_hbm.at[wid], dst_v)
        plsc.subcore_barrier()

        for b in range(_NBUF):
            pltpu.async_copy(ones_v, acc_sh.at[dst_v.at[b]], ssems[b],
                             add=True)

        def body(g, carry):
            for b in range(_NBUF):
                m = (g + 1) * _NBUF + b
                pltpu.make_async_copy(
                    ones_v, acc_sh.at[dst_v.at[m - _NBUF]], ssems[b]).wait()
                pltpu.async_copy(ones_v, acc_sh.at[dst_v.at[m]], ssems[b],
                                 add=True)
            return carry

        lax.fori_loop(0, iters // _NBUF - 1, body, 0)
        for b in range(_NBUF):
            pltpu.make_async_copy(
                ones_v, acc_sh.at[dst_v.at[iters - _NBUF + b]],
                ssems[b]).wait()
        plsc.subcore_barrier()

        def out_copy(off):
            pltpu.sync_copy(acc_sh.at[pl.ds(off, _CH_ROWS)],
                            out_hbm.at[ci, pl.ds(off, _CH_ROWS)])

        _row_chunks(n, si, out_copy)

    return deg


_NBUF = 5
_LAG = 1
_TRASH = 400


def _sc_agg(n, h, iters, chunk):
    mesh = plsc.VectorSubcoreMesh(core_axis_name="c", subcore_axis_name="s")

    @functools.partial(
        pl.kernel,
        out_type=jax.ShapeDtypeStruct((_NC, n, h), _F32),
        mesh=mesh,
        compiler_params=pltpu.CompilerParams(use_tc_tiling_on_sc=False),
        scratch_types=[
            pltpu.VMEM((iters, chunk), jnp.int32),
            pltpu.VMEM((iters, chunk), jnp.int32),
            pltpu.VMEM((_NBUF, chunk, h), _F32),
            pltpu.VMEM_SHARED((n + _TRASH, h), _F32),
        ] + [pltpu.SemaphoreType.DMA] * (2 * _NBUF),
    )
    def agg(g_hbm, src_hbm, dst_hbm, out_hbm,
            src_v, dst_v, rows_v, acc_sh, *sems):
        gsems = sems[:_NBUF]
        ssems = sems[_NBUF:]
        ci = lax.axis_index("c")
        si = lax.axis_index("s")
        wid = ci * _NS + si

        pltpu.async_copy(src_hbm.at[wid], src_v, ssems[0])
        pltpu.async_copy(dst_hbm.at[wid], dst_v, ssems[1])

        n_chunks = n // _CH_ROWS
        max_k = (n_chunks + _NS - 1) // _NS

        def init_fire(cid, k):
            pltpu.async_copy(g_hbm.at[pl.ds(cid * _CH_ROWS, _CH_ROWS)],
                             acc_sh.at[pl.ds(cid * _CH_ROWS, _CH_ROWS)],
                             ssems[2 + k])

        def init_drain(cid, k):
            pltpu.make_async_copy(
                g_hbm.at[pl.ds(cid * _CH_ROWS, _CH_ROWS)],
                acc_sh.at[pl.ds(cid * _CH_ROWS, _CH_ROWS)],
                ssems[2 + k]).wait()

        for k in range(max_k):
            cid = si + _NS * k
            if (k + 1) * _NS <= n_chunks:
                init_fire(cid, k)
            else:
                @pl.when(cid < n_chunks)
                def _():
                    init_fire(cid, k)

        pltpu.make_async_copy(src_hbm.at[wid], src_v, ssems[0]).wait()
        pltpu.make_async_copy(dst_hbm.at[wid], dst_v, ssems[1]).wait()

        for b in range(_NBUF):
            pltpu.async_copy(g_hbm.at[src_v.at[b]], rows_v.at[b], gsems[b])

        for k in range(max_k):
            cid = si + _NS * k
            if (k + 1) * _NS <= n_chunks:
                init_drain(cid, k)
            else:
                @pl.when(cid < n_chunks)
                def _():
                    init_drain(cid, k)

        plsc.subcore_barrier()

        def body(g, carry):
            for b in range(_NBUF):
                m = g * _NBUF + b
                pltpu.make_async_copy(
                    g_hbm.at[src_v.at[m]], rows_v.at[b], gsems[b]).wait()
                pltpu.async_copy(
                    rows_v.at[b], acc_sh.at[dst_v.at[m]], ssems[b], add=True)
                bp = (b - _LAG) % _NBUF
                mp = m - _LAG

                @pl.when(mp >= 0)
                def _():
                    pltpu.make_async_copy(
                        rows_v.at[bp], acc_sh.at[dst_v.at[mp]],
                        ssems[bp]).wait()

                    @pl.when(mp + _NBUF < iters)
                    def _():
                        pltpu.async_copy(
                            g_hbm.at[src_v.at[mp + _NBUF]], rows_v.at[bp],
                            gsems[bp])

            return carry

        lax.fori_loop(0, iters // _NBUF, body, 0)
        for k in range(_LAG):
            bl = (iters - _LAG + k) % _NBUF
            pltpu.make_async_copy(
                rows_v.at[bl], acc_sh.at[dst_v.at[iters - _LAG + k]],
                ssems[bl]).wait()
        plsc.subcore_barrier()

        def out_copy(off):
            pltpu.sync_copy(acc_sh.at[pl.ds(off, _CH_ROWS)],
                            out_hbm.at[ci, pl.ds(off, _CH_ROWS)])

        _row_chunks(n, si, out_copy)

    return agg


def _dot(a, b, precision=jax.lax.Precision.HIGHEST):
    return jax.lax.dot_general(a, b, (((1,), (0,)), ((), ())),
                               preferred_element_type=_F32,
                               precision=precision)


def _tc_first(deg_parts, x, w1, rb):
    n, d_in = x.shape
    hdim = w1.shape[1]

    def kfn(dp_ref, x_ref, w_ref, dinv_ref, g_ref):
        dp = dp_ref[0] + dp_ref[1]
        deg = jnp.sum(dp, axis=1, keepdims=True) + 1.0
        dinv = lax.rsqrt(deg)
        dinv_ref[...] = dinv
        g_ref[...] = dinv * _dot(x_ref[...], w_ref[...])

    return pl.pallas_call(
        kfn,
        grid=(n // rb,),
        in_specs=[
            pl.BlockSpec((2, rb, _DEGW), lambda i: (0, i, 0)),
            pl.BlockSpec((rb, d_in), lambda i: (i, 0)),
            pl.BlockSpec((d_in, hdim), lambda i: (0, 0)),
        ],
        out_specs=[
            pl.BlockSpec((rb, 1), lambda i: (i, 0)),
            pl.BlockSpec((rb, hdim), lambda i: (i, 0)),
        ],
        out_shape=[
            jax.ShapeDtypeStruct((n, 1), _F32),
            jax.ShapeDtypeStruct((n, hdim), _F32),
        ],
    )(deg_parts, x, w1)


def _tc_mid(s_parts, g_prev, dinv, w_next, b, rb):
    n, hdim = g_prev.shape
    hout = w_next.shape[1]

    def kfn(sp_ref, gp_ref, dinv_ref, w_ref, b_ref, out_ref):
        s = sp_ref[0] + sp_ref[1] - gp_ref[...]
        dv = dinv_ref[...]
        hact = jnp.maximum(dv * s + b_ref[...], 0.0)
        out_ref[...] = dv * _dot(hact, w_ref[...])

    return pl.pallas_call(
        kfn,
        grid=(n // rb,),
        in_specs=[
            pl.BlockSpec((2, rb, hdim), lambda i: (0, i, 0)),
            pl.BlockSpec((rb, hdim), lambda i: (i, 0)),
            pl.BlockSpec((rb, 1), lambda i: (i, 0)),
            pl.BlockSpec((hdim, hout), lambda i: (0, 0)),
            pl.BlockSpec((1, hdim), lambda i: (0, 0)),
        ],
        out_specs=pl.BlockSpec((rb, hout), lambda i: (i, 0)),
        out_shape=jax.ShapeDtypeStruct((n, hout), _F32),
    )(s_parts, g_prev, dinv, w_next, b)


def _tc_last_g(s_parts, g_prev, dinv, b, rb):
    n, hdim = g_prev.shape

    def kfn(sp_ref, gp_ref, dinv_ref, b_ref, out_ref):
        s = sp_ref[0] + sp_ref[1] - gp_ref[...]
        dv = dinv_ref[...]
        out_ref[...] = dv * jnp.maximum(dv * s + b_ref[...], 0.0)

    return pl.pallas_call(
        kfn,
        grid=(n // rb,),
        in_specs=[
            pl.BlockSpec((2, rb, hdim), lambda i: (0, i, 0)),
            pl.BlockSpec((rb, hdim), lambda i: (i, 0)),
            pl.BlockSpec((rb, 1), lambda i: (i, 0)),
            pl.BlockSpec((1, hdim), lambda i: (0, 0)),
        ],
        out_specs=pl.BlockSpec((rb, hdim), lambda i: (i, 0)),
        out_shape=jax.ShapeDtypeStruct((n, hdim), _F32),
    )(s_parts, g_prev, dinv, b)


def _tc_readout(s_parts, g_prev, dinv, w4, b4, batch3, nseg, rb):
    n, hdim = g_prev.shape
    dout = w4.shape[1]
    nblocks = n // rb

    def kfn(sp_ref, gp_ref, dinv_ref, w_ref, b_ref, batch_ref, out_ref,
            sums_sc, cnt_sc):
        i = pl.program_id(0)

        @pl.when(i == 0)
        def _():
            sums_sc[...] = jnp.zeros_like(sums_sc)
            cnt_sc[...] = jnp.zeros_like(cnt_sc)

        u = dinv_ref[...] * (sp_ref[0] + sp_ref[1] - gp_ref[...])
        h4 = _dot(u, w_ref[...]) + b_ref[...]
        bvec = batch_ref[0]
        onehot_t = jnp.where(
            lax.broadcasted_iota(jnp.int32, (nseg, rb), 0) == bvec, 1.0, 0.0)
        sums_sc[...] += _dot(onehot_t, h4)
        cnt_sc[...] += jnp.sum(onehot_t, axis=1, keepdims=True)

        @pl.when(i == nblocks - 1)
        def _():
            mean = sums_sc[...] / jnp.maximum(cnt_sc[...], 1.0)
            m = jnp.max(mean, axis=1, keepdims=True)
            lse = jnp.log(jnp.sum(jnp.exp(mean - m), axis=1, keepdims=True)) + m
            out_ref[...] = mean - lse

    return pl.pallas_call(
        kfn,
        grid=(nblocks,),
        in_specs=[
            pl.BlockSpec((2, rb, hdim), lambda i: (0, i, 0)),
            pl.BlockSpec((rb, hdim), lambda i: (i, 0)),
            pl.BlockSpec((rb, 1), lambda i: (i, 0)),
            pl.BlockSpec((hdim, dout), lambda i: (0, 0)),
            pl.BlockSpec((1, dout), lambda i: (0, 0)),
            pl.BlockSpec((1, 1, rb), lambda i: (i, 0, 0)),
        ],
        out_specs=pl.BlockSpec((nseg, dout), lambda i: (0, 0)),
        out_shape=jax.ShapeDtypeStruct((nseg, dout), _F32),
        scratch_shapes=[
            pltpu.VMEM((nseg, dout), _F32),
            pltpu.VMEM((nseg, 1), _F32),
        ],
    )(s_parts, g_prev, dinv, w4, b4, batch3)


def kernel(x, edge_index, batch, W1, b1, W2, b2, W3, b3, W4, b4):
    n, d_in = x.shape
    e = edge_index.shape[1]
    hdim = W1.shape[1]
    nseg = 64
    rb = 2000

    per_w = e // _NW
    iters = -(-per_w // (_CHUNK * _NBUF)) * _NBUF
    e_pad = _NW * iters * _CHUNK - e
    if e_pad:
        src_p = jnp.concatenate([edge_index[0],
                                 jnp.zeros((e_pad,), jnp.int32)])
        dst_p = jnp.concatenate([edge_index[1],
                                 n + (jnp.arange(e_pad, dtype=jnp.int32)
                                      % _TRASH)])
    else:
        src_p, dst_p = edge_index[0], edge_index[1]
    src3 = src_p.reshape(_NW, iters, _CHUNK)
    dst3 = dst_p.reshape(_NW, iters, _CHUNK)
    ones_h = jnp.ones((_CHUNK, _DEGW), _F32)
    zeros_h = jnp.zeros((_CH_ROWS, _DEGW), _F32)
    batch3 = batch.reshape(n // rb, 1, rb)

    deg_parts = _sc_deg(n, iters, _CHUNK)(ones_h, zeros_h, dst3)
    dinv, g1 = _tc_first(deg_parts, x, W1, rb)

    agg = _sc_agg(n, hdim, iters, _CHUNK)
    s1 = agg(g1, src3, dst3)
    g2 = _tc_mid(s1, g1, dinv, W2, b1.reshape(1, -1), rb)
    s2 = agg(g2, src3, dst3)
    g3 = _tc_mid(s2, g2, dinv, W3, b2.reshape(1, -1), rb)
    s3 = agg(g3, src3, dst3)
    g4 = _tc_last_g(s3, g3, dinv, b3.reshape(1, -1), rb)
    s4 = agg(g4, src3, dst3)
    return _tc_readout(s4, g4, dinv, W4, b4.reshape(1, -1), batch3, nseg, rb)


# speedup vs baseline: 1.0857x; 1.0030x over previous
import functools

import jax
import jax.numpy as jnp
from jax import lax
from jax.experimental import pallas as pl
from jax.experimental.pallas import tpu as pltpu
from jax.experimental.pallas import tpu_sc as plsc

_NC = 2
_NS = 16
_NW = _NC * _NS

_F32 = jnp.float32
_CHUNK = 80
_DEGW = 8
_CH_ROWS = 400


def _row_chunks(n, si, fn):
    n_chunks = n // _CH_ROWS
    max_per_tile = (n_chunks + _NS - 1) // _NS
    for k in range(max_per_tile):
        cid = si + _NS * k
        if (k + 1) * _NS <= n_chunks:
            fn(cid * _CH_ROWS)
        else:
            @pl.when(cid < n_chunks)
            def _():
                fn(cid * _CH_ROWS)


def _sc_deg(n, iters, chunk):
    mesh = plsc.VectorSubcoreMesh(core_axis_name="c", subcore_axis_name="s")

    @functools.partial(
        pl.kernel,
        out_type=jax.ShapeDtypeStruct((_NC, n, _DEGW), _F32),
        mesh=mesh,
        compiler_params=pltpu.CompilerParams(use_tc_tiling_on_sc=False),
        scratch_types=[
            pltpu.VMEM((iters, chunk), jnp.int32),
            pltpu.VMEM((chunk, _DEGW), _F32),
            pltpu.VMEM_SHARED((n + _TRASH, _DEGW), _F32),
        ] + [pltpu.SemaphoreType.DMA] * _NBUF,
    )
    def deg(ones_hbm, zeros_hbm, dst_hbm, out_hbm, dst_v, ones_v, acc_sh,
            *ssems):
        ci = lax.axis_index("c")
        si = lax.axis_index("s")
        wid = ci * _NS + si

        pltpu.async_copy(dst_hbm.at[wid], dst_v, ssems[0])
        pltpu.async_copy(ones_hbm, ones_v, ssems[1])

        n_chunks = n // _CH_ROWS
        max_k = (n_chunks + _NS - 1) // _NS

        def z_fire(cid, k):
            pltpu.async_copy(zeros_hbm,
                             acc_sh.at[pl.ds(cid * _CH_ROWS, _CH_ROWS)],
                             ssems[2 + k])

        def z_drain(cid, k):
            pltpu.make_async_copy(
                zeros_hbm, acc_sh.at[pl.ds(cid * _CH_ROWS, _CH_ROWS)],
                ssems[2 + k]).wait()

        for k in range(max_k):
            cid = si + _NS * k
            if (k + 1) * _NS <= n_chunks:
                z_fire(cid, k)
            else:
                @pl.when(cid < n_chunks)
                def _():
                    z_fire(cid, k)

        pltpu.make_async_copy(dst_hbm.at[wid], dst_v, ssems[0]).wait()
        pltpu.make_async_copy(ones_hbm, ones_v, ssems[1]).wait()

        for k in range(max_k):
            cid = si + _NS * k
            if (k + 1) * _NS <= n_chunks:
                z_drain(cid, k)
            else:
                @pl.when(cid < n_chunks)
                def _():
                    z_drain(cid, k)

        plsc.subcore_barrier()

        for b in range(_NBUF):
            pltpu.async_copy(ones_v, acc_sh.at[dst_v.at[b]], ssems[b],
                             add=True)

        def body(g, carry):
            for b in range(_NBUF):
                m = (g + 1) * _NBUF + b
                pltpu.make_async_copy(
                    ones_v, acc_sh.at[dst_v.at[m - _NBUF]], ssems[b]).wait()
                pltpu.async_copy(ones_v, acc_sh.at[dst_v.at[m]], ssems[b],
                                 add=True)
            return carry

        lax.fori_loop(0, iters // _NBUF - 1, body, 0)
        for b in range(_NBUF):
            pltpu.make_async_copy(
                ones_v, acc_sh.at[dst_v.at[iters - _NBUF + b]],
                ssems[b]).wait()
        plsc.subcore_barrier()

        def out_copy(off):
            pltpu.sync_copy(acc_sh.at[pl.ds(off, _CH_ROWS)],
                            out_hbm.at[ci, pl.ds(off, _CH_ROWS)])

        _row_chunks(n, si, out_copy)

    return deg


_NBUF = 5
_LAG = 1
_TRASH = 400


def _sc_agg(n, h, iters, chunk):
    mesh = plsc.VectorSubcoreMesh(core_axis_name="c", subcore_axis_name="s")

    @functools.partial(
        pl.kernel,
        out_type=jax.ShapeDtypeStruct((_NC, n, h), _F32),
        mesh=mesh,
        compiler_params=pltpu.CompilerParams(use_tc_tiling_on_sc=False),
        scratch_types=[
            pltpu.VMEM((iters, chunk), jnp.int32),
            pltpu.VMEM((iters, chunk), jnp.int32),
            pltpu.VMEM((_NBUF, chunk, h), _F32),
            pltpu.VMEM_SHARED((n + _TRASH, h), _F32),
        ] + [pltpu.SemaphoreType.DMA] * (2 * _NBUF),
    )
    def agg(g_hbm, src_hbm, dst_hbm, out_hbm,
            src_v, dst_v, rows_v, acc_sh, *sems):
        gsems = sems[:_NBUF]
        ssems = sems[_NBUF:]
        ci = lax.axis_index("c")
        si = lax.axis_index("s")
        wid = ci * _NS + si

        pltpu.async_copy(src_hbm.at[wid], src_v, ssems[0])
        pltpu.async_copy(dst_hbm.at[wid], dst_v, ssems[1])

        n_chunks = n // _CH_ROWS
        max_k = (n_chunks + _NS - 1) // _NS

        def init_fire(cid, k):
            pltpu.async_copy(g_hbm.at[pl.ds(cid * _CH_ROWS, _CH_ROWS)],
                             acc_sh.at[pl.ds(cid * _CH_ROWS, _CH_ROWS)],
                             ssems[2 + k])

        def init_drain(cid, k):
            pltpu.make_async_copy(
                g_hbm.at[pl.ds(cid * _CH_ROWS, _CH_ROWS)],
                acc_sh.at[pl.ds(cid * _CH_ROWS, _CH_ROWS)],
                ssems[2 + k]).wait()

        for k in range(max_k):
            cid = si + _NS * k
            if (k + 1) * _NS <= n_chunks:
                init_fire(cid, k)
            else:
                @pl.when(cid < n_chunks)
                def _():
                    init_fire(cid, k)

        pltpu.make_async_copy(src_hbm.at[wid], src_v, ssems[0]).wait()
        pltpu.make_async_copy(dst_hbm.at[wid], dst_v, ssems[1]).wait()

        for b in range(_NBUF):
            pltpu.async_copy(g_hbm.at[src_v.at[b]], rows_v.at[b], gsems[b])

        for k in range(max_k):
            cid = si + _NS * k
            if (k + 1) * _NS <= n_chunks:
                init_drain(cid, k)
            else:
                @pl.when(cid < n_chunks)
                def _():
                    init_drain(cid, k)

        plsc.subcore_barrier()

        def body(g, carry):
            for b in range(_NBUF):
                m = g * _NBUF + b
                pltpu.make_async_copy(
                    g_hbm.at[src_v.at[m]], rows_v.at[b], gsems[b]).wait()
                pltpu.async_copy(
                    rows_v.at[b], acc_sh.at[dst_v.at[m]], ssems[b], add=True)
                bp = (b - _LAG) % _NBUF
                mp = m - _LAG

                @pl.when(mp >= 0)
                def _():
                    pltpu.make_async_copy(
                        rows_v.at[bp], acc_sh.at[dst_v.at[mp]],
                        ssems[bp]).wait()

                    @pl.when(mp + _NBUF < iters)
                    def _():
                        pltpu.async_copy(
                            g_hbm.at[src_v.at[mp + _NBUF]], rows_v.at[bp],
                            gsems[bp])

            return carry

        lax.fori_loop(0, iters // _NBUF, body, 0)
        for k in range(_LAG):
            bl = (iters - _LAG + k) % _NBUF
            pltpu.make_async_copy(
                rows_v.at[bl], acc_sh.at[dst_v.at[iters - _LAG + k]],
                ssems[bl]).wait()
        plsc.subcore_barrier()

        def out_copy(off):
            pltpu.sync_copy(acc_sh.at[pl.ds(off, _CH_ROWS)],
                            out_hbm.at[ci, pl.ds(off, _CH_ROWS)])

        _row_chunks(n, si, out_copy)

    return agg


def _dot(a, b, precision=jax.lax.Precision.HIGHEST):
    return jax.lax.dot_general(a, b, (((1,), (0,)), ((), ())),
                               preferred_element_type=_F32,
                               precision=precision)


def _tc_first(deg_parts, x, w1, rb):
    n, d_in = x.shape
    hdim = w1.shape[1]

    def kfn(dp_ref, x_ref, w_ref, dinv_ref, g_ref):
        dp = dp_ref[0] + dp_ref[1]
        deg = jnp.sum(dp, axis=1, keepdims=True) + 1.0
        dinv = lax.rsqrt(deg)
        dinv_ref[...] = dinv
        g_ref[...] = dinv * _dot(x_ref[...], w_ref[...])

    return pl.pallas_call(
        kfn,
        grid=(n // rb,),
        in_specs=[
            pl.BlockSpec((2, rb, _DEGW), lambda i: (0, i, 0)),
            pl.BlockSpec((rb, d_in), lambda i: (i, 0)),
            pl.BlockSpec((d_in, hdim), lambda i: (0, 0)),
        ],
        out_specs=[
            pl.BlockSpec((rb, 1), lambda i: (i, 0)),
            pl.BlockSpec((rb, hdim), lambda i: (i, 0)),
        ],
        out_shape=[
            jax.ShapeDtypeStruct((n, 1), _F32),
            jax.ShapeDtypeStruct((n, hdim), _F32),
        ],
    )(deg_parts, x, w1)


def _tc_mid(s_parts, g_prev, dinv, w_next, b, rb):
    n, hdim = g_prev.shape
    hout = w_next.shape[1]

    def kfn(sp_ref, gp_ref, dinv_ref, w_ref, b_ref, out_ref):
        s = sp_ref[0] + sp_ref[1] - gp_ref[...]
        dv = dinv_ref[...]
        hact = jnp.maximum(dv * s + b_ref[...], 0.0)
        out_ref[...] = dv * _dot(hact, w_ref[...])

    return pl.pallas_call(
        kfn,
        grid=(n // rb,),
        in_specs=[
            pl.BlockSpec((2, rb, hdim), lambda i: (0, i, 0)),
            pl.BlockSpec((rb, hdim), lambda i: (i, 0)),
            pl.BlockSpec((rb, 1), lambda i: (i, 0)),
            pl.BlockSpec((hdim, hout), lambda i: (0, 0)),
            pl.BlockSpec((1, hdim), lambda i: (0, 0)),
        ],
        out_specs=pl.BlockSpec((rb, hout), lambda i: (i, 0)),
        out_shape=jax.ShapeDtypeStruct((n, hout), _F32),
    )(s_parts, g_prev, dinv, w_next, b)


def _tc_last_g(s_parts, g_prev, dinv, b, rb):
    n, hdim = g_prev.shape

    def kfn(sp_ref, gp_ref, dinv_ref, b_ref, out_ref):
        s = sp_ref[0] + sp_ref[1] - gp_ref[...]
        dv = dinv_ref[...]
        out_ref[...] = dv * jnp.maximum(dv * s + b_ref[...], 0.0)

    return pl.pallas_call(
        kfn,
        grid=(n // rb,),
        in_specs=[
            pl.BlockSpec((2, rb, hdim), lambda i: (0, i, 0)),
            pl.BlockSpec((rb, hdim), lambda i: (i, 0)),
            pl.BlockSpec((rb, 1), lambda i: (i, 0)),
            pl.BlockSpec((1, hdim), lambda i: (0, 0)),
        ],
        out_specs=pl.BlockSpec((rb, hdim), lambda i: (i, 0)),
        out_shape=jax.ShapeDtypeStruct((n, hdim), _F32),
    )(s_parts, g_prev, dinv, b)


def _tc_readout(s_parts, g_prev, dinv, w4, b4, batch3, nseg, rb):
    n, hdim = g_prev.shape
    dout = w4.shape[1]
    nblocks = n // rb

    def kfn(sp_ref, gp_ref, dinv_ref, w_ref, b_ref, batch_ref, out_ref,
            sums_sc, cnt_sc):
        i = pl.program_id(0)

        @pl.when(i == 0)
        def _():
            sums_sc[...] = jnp.zeros_like(sums_sc)
            cnt_sc[...] = jnp.zeros_like(cnt_sc)

        u = dinv_ref[...] * (sp_ref[0] + sp_ref[1] - gp_ref[...])
        h4 = _dot(u, w_ref[...]) + b_ref[...]
        bvec = batch_ref[0]
        onehot_t = jnp.where(
            lax.broadcasted_iota(jnp.int32, (nseg, rb), 0) == bvec, 1.0, 0.0)
        sums_sc[...] += _dot(onehot_t, h4)
        cnt_sc[...] += jnp.sum(onehot_t, axis=1, keepdims=True)

        @pl.when(i == nblocks - 1)
        def _():
            mean = sums_sc[...] / jnp.maximum(cnt_sc[...], 1.0)
            m = jnp.max(mean, axis=1, keepdims=True)
            lse = jnp.log(jnp.sum(jnp.exp(mean - m), axis=1, keepdims=True)) + m
            out_ref[...] = mean - lse

    return pl.pallas_call(
        kfn,
        grid=(nblocks,),
        in_specs=[
            pl.BlockSpec((2, rb, hdim), lambda i: (0, i, 0)),
            pl.BlockSpec((rb, hdim), lambda i: (i, 0)),
            pl.BlockSpec((rb, 1), lambda i: (i, 0)),
            pl.BlockSpec((hdim, dout), lambda i: (0, 0)),
            pl.BlockSpec((1, dout), lambda i: (0, 0)),
            pl.BlockSpec((1, 1, rb), lambda i: (i, 0, 0)),
        ],
        out_specs=pl.BlockSpec((nseg, dout), lambda i: (0, 0)),
        out_shape=jax.ShapeDtypeStruct((nseg, dout), _F32),
        scratch_shapes=[
            pltpu.VMEM((nseg, dout), _F32),
            pltpu.VMEM((nseg, 1), _F32),
        ],
    )(s_parts, g_prev, dinv, w4, b4, batch3)


def kernel(x, edge_index, batch, W1, b1, W2, b2, W3, b3, W4, b4):
    n, d_in = x.shape
    e = edge_index.shape[1]
    hdim = W1.shape[1]
    nseg = 64
    rb = 2000

    per_w = e // _NW
    iters = -(-per_w // (_CHUNK * _NBUF)) * _NBUF
    e_pad = _NW * iters * _CHUNK - e
    if e_pad:
        src_p = jnp.concatenate([edge_index[0],
                                 jnp.zeros((e_pad,), jnp.int32)])
        dst_p = jnp.concatenate([edge_index[1],
                                 n + (jnp.arange(e_pad, dtype=jnp.int32)
                                      % _TRASH)])
    else:
        src_p, dst_p = edge_index[0], edge_index[1]
    src3 = src_p.reshape(_NW, iters, _CHUNK)
    dst3 = dst_p.reshape(_NW, iters, _CHUNK)
    ones_h = jnp.ones((_CHUNK, _DEGW), _F32)
    zeros_h = jnp.zeros((_CH_ROWS, _DEGW), _F32)
    batch3 = batch.reshape(n // rb, 1, rb)

    deg_parts = _sc_deg(n, iters, _CHUNK)(ones_h, zeros_h, dst3)
    dinv, g1 = _tc_first(deg_parts, x, W1, rb)

    agg = _sc_agg(n, hdim, iters, _CHUNK)
    s1 = agg(g1, src3, dst3)
    g2 = _tc_mid(s1, g1, dinv, W2, b1.reshape(1, -1), rb)
    s2 = agg(g2, src3, dst3)
    g3 = _tc_mid(s2, g2, dinv, W3, b2.reshape(1, -1), rb)
    s3 = agg(g3, src3, dst3)
    g4 = _tc_last_g(s3, g3, dinv, b3.reshape(1, -1), rb)
    s4 = agg(g4, src3, dst3)
    return _tc_readout(s4, g4, dinv, W4, b4.reshape(1, -1), batch3, nseg, rb)
